# baseline (speedup 1.0000x reference)
import functools

import jax
import jax.numpy as jnp
from jax import lax
from jax.experimental import pallas as pl
from jax.experimental.pallas import tpu as pltpu
from jax.experimental.pallas import tpu_sc as plsc

CL = 4


def _keep_body(len_keep, nrow_ref, ncol_ref, kx_ref, mask_ref):
    L = nrow_ref.shape[-1]
    nvars = mask_ref.shape[-1]
    nj = nrow_ref[0]
    nl = ncol_ref[0]
    jidx = lax.broadcasted_iota(jnp.int32, (L, L), 1)
    lg = lax.broadcasted_iota(jnp.int32, (L, L), 0)
    cnt = (nj < nl) | ((nj == nl) & (jidx < lg))
    rank = jnp.sum(cnt.astype(jnp.int32), axis=1, keepdims=True)
    keep = (rank < len_keep).astype(jnp.float32)
    kx_ref[0] = jnp.broadcast_to(keep, (L, 128))
    mask_ref[0] = jnp.broadcast_to(1.0 - keep, (L, nvars))


def _sc_body(nvars, D, NS, xb_hbm, kx_hbm, out_hbm,
             vin, vout, kbuf, si0, si1, sk0, sk1, so0, so1):
    L = kx_hbm.shape[1]
    NCH = L // CL
    b = lax.axis_index("c") * NS + lax.axis_index("s")
    sems_in = (si0, si1)
    sems_k = (sk0, sk1)
    sems_out = (so0, so1)

    def in_data(c, rb):
        return pltpu.make_async_copy(
            xb_hbm.at[b, pl.ds(c * CL, CL)], vin.at[rb], sems_in[rb])

    def in_keep(c, rb):
        return pltpu.make_async_copy(
            kx_hbm.at[b, pl.ds(c * CL, CL)], kbuf.at[rb], sems_k[rb])

    def out_data(c, rb):
        return pltpu.make_async_copy(
            vout.at[rb], out_hbm.at[b, pl.ds(c * CL, CL)], sems_out[rb])

    def compute(rb):
        for l in range(CL):
            kv = kbuf[rb, l, pl.ds(0, 16)]
            for v in range(nvars):
                for k in range(D // 16):
                    sl = pl.ds(k * 16, 16)
                    vout[rb, l, v, sl] = vin[rb, l, v, sl] * kv

    in_data(0, 0).start()
    in_keep(0, 0).start()

    def body(i, carry):
        c0 = i * 2
        in_data(c0 + 1, 1).start()
        in_keep(c0 + 1, 1).start()

        @pl.when(c0 >= 2)
        def _():
            out_data(c0 - 2, 0).wait()

        in_data(c0, 0).wait()
        in_keep(c0, 0).wait()
        compute(0)
        out_data(c0, 0).start()

        @pl.when(c0 + 2 < NCH)
        def _():
            in_data(c0 + 2, 0).start()
            in_keep(c0 + 2, 0).start()

        @pl.when(c0 >= 2)
        def _():
            out_data(c0 - 1, 1).wait()

        in_data(c0 + 1, 1).wait()
        in_keep(c0 + 1, 1).wait()
        compute(1)
        out_data(c0 + 1, 1).start()
        return carry

    lax.fori_loop(0, NCH // 2, body, 0)
    out_data(NCH - 2, 0).wait()
    out_data(NCH - 1, 1).wait()


@jax.jit
def kernel(xb):
    bs, L, nvars, D = xb.shape
    len_keep = int(L * (1 - 0.15))
    noise = jax.random.uniform(jax.random.key(42), (bs, L), dtype=jnp.float32)
    nrow = noise.reshape(bs, 1, L)
    ncol = noise.reshape(bs, L, 1)

    kx, mask = pl.pallas_call(
        functools.partial(_keep_body, len_keep),
        grid=(bs,),
        in_specs=[
            pl.BlockSpec((1, 1, L), lambda b: (b, 0, 0)),
            pl.BlockSpec((1, L, 1), lambda b: (b, 0, 0)),
        ],
        out_specs=[
            pl.BlockSpec((1, L, 128), lambda b: (b, 0, 0)),
            pl.BlockSpec((1, L, nvars), lambda b: (b, 0, 0)),
        ],
        out_shape=[
            jax.ShapeDtypeStruct((bs, L, 128), jnp.float32),
            jax.ShapeDtypeStruct((bs, L, nvars), jnp.float32),
        ],
    )(nrow, ncol)

    NC, NS = 2, 16
    assert NC * NS == bs
    mesh = plsc.VectorSubcoreMesh(
        core_axis_name="c", subcore_axis_name="s", num_cores=NC)
    sc_fn = functools.partial(
        pl.kernel,
        mesh=mesh,
        out_type=jax.ShapeDtypeStruct((bs, L, nvars, D), jnp.float32),
        scratch_types=[
            pltpu.VMEM((2, CL, nvars, D), jnp.float32),
            pltpu.VMEM((2, CL, nvars, D), jnp.float32),
            pltpu.VMEM((2, CL, 128), jnp.float32),
            pltpu.SemaphoreType.DMA,
            pltpu.SemaphoreType.DMA,
            pltpu.SemaphoreType.DMA,
            pltpu.SemaphoreType.DMA,
            pltpu.SemaphoreType.DMA,
            pltpu.SemaphoreType.DMA,
        ],
    )(functools.partial(_sc_body, nvars, D, NS))
    x_masked = sc_fn(xb, kx)
    return x_masked, mask
